# baseline (speedup 1.0000x reference)
import functools
import jax
import jax.numpy as jnp
from jax import lax
from jax.experimental import pallas as pl
from jax.experimental.pallas import tpu as pltpu
from jax.experimental.pallas import tpu_sc as plsc

N = 100000
E = 1600000
NC, NS = 2, 16
NW = NC * NS
NPAD = 100096
SUB = NPAD // NS
ER = E // 128
KJD = 16
KJM = 8
GB = 1000


def _shard(kj):
    nch = ER // kj
    return nch, ER - nch * kj, nch // NW, nch - (nch // NW) * NW


def _worker_chunks(wid, kj):
    nch_tot, _, base, extra = _shard(kj)
    c0 = wid * base + jnp.minimum(wid, extra)
    nch = base + (wid < extra).astype(jnp.int32)
    return c0, nch


def _mesh():
    return plsc.VectorSubcoreMesh(
        core_axis_name="c", subcore_axis_name="s",
        num_cores=NC, num_subcores=NS)


_SC_PARAMS = pltpu.CompilerParams(use_tc_tiling_on_sc=False,
                                 needs_layout_passes=False)


DBIG = KJD * 128
DNCH, DTAIL, _, _ = 12500 // KJD, 12500 - (12500 // KJD) * KJD, 0, 0


def _deg_body(e3_hbm, zeros_hbm, out_hbm, ev, dstv, ones_v, buf_v,
              acc_sh, sem):
    cid = lax.axis_index("c")
    sid = lax.axis_index("s")
    wid = sid * NC + cid
    for i in range(DBIG // 16):
        ones_v[pl.ds(i * 16, 16)] = jnp.full((16,), 1.0, jnp.float32)
    pltpu.sync_copy(zeros_hbm.at[pl.ds(sid * SUB, SUB)], buf_v)
    pltpu.sync_copy(buf_v, acc_sh.at[pl.ds(sid * SUB, SUB)])
    plsc.subcore_barrier()
    c0, nch = _worker_chunks(wid, KJD)

    def step(t, carry):
        p = t % 2
        pltpu.sync_copy(e3_hbm.at[pl.ds((c0 + t) * KJD, KJD)], ev)

        @pl.when(t >= 2)
        def _drain():
            pltpu.make_async_copy(
                zeros_hbm.at[pl.ds(0, DBIG)], ones_v, sem).wait()

        for j in range(KJD):
            for k in range(128 // 16):
                dstv[p, pl.ds(j * 128 + k * 16, 16)] = \
                    ev[j, 1, pl.ds(k * 16, 16)]
        pltpu.async_copy(ones_v, acc_sh.at[dstv.at[p]], sem, add=True)
        return carry

    lax.fori_loop(0, nch, step, 0)
    for _ in range(2):
        pltpu.make_async_copy(zeros_hbm.at[pl.ds(0, DBIG)], ones_v, sem).wait()

    @pl.when(wid == NW - 1)
    def _tail():
        pltpu.sync_copy(e3_hbm.at[pl.ds(DNCH * KJD, DTAIL)],
                        ev.at[pl.ds(0, DTAIL)])
        for j in range(DTAIL):
            pltpu.sync_copy(ones_v.at[pl.ds(0, 128)],
                            acc_sh.at[ev.at[j, 1]], add=True)

    plsc.subcore_barrier()
    pltpu.sync_copy(acc_sh.at[pl.ds(sid * SUB, SUB)], buf_v)
    pltpu.sync_copy(buf_v, out_hbm.at[pl.ds(cid * NPAD + sid * SUB, SUB)])


def _deg_call(e3, zeros1):
    return pl.kernel(
        _deg_body,
        out_type=jax.ShapeDtypeStruct((NC * NPAD,), jnp.float32),
        mesh=_mesh(),
        scratch_types=[
            pltpu.VMEM((KJD, 2, 128), jnp.int32),
            pltpu.VMEM((2, DBIG), jnp.int32),
            pltpu.VMEM((DBIG,), jnp.float32),
            pltpu.VMEM((SUB,), jnp.float32),
            pltpu.VMEM_SHARED((NPAD,), jnp.float32),
            pltpu.SemaphoreType.DMA,
        ],
        compiler_params=_SC_PARAMS,
    )(e3, zeros1)


MBIG = KJM * 128
MNCH = ER // KJM
MTAIL = ER - MNCH * KJM


def _msg_body(e3_hbm, up_hbm, zeros_hbm,
              out0_hbm, out1_hbm,
              ev, upt_v, val0_v, val1_v, dstv, tval0_v, tval1_v, buf_v,
              acc0_sh, acc1_sh, sem):
    cid = lax.axis_index("c")
    sid = lax.axis_index("s")
    wid = sid * NC + cid
    zb_hbm = zeros_hbm.at[pl.ds(0, MBIG)]
    pltpu.sync_copy(up_hbm.at[pl.ds(0, N)], upt_v)
    pltpu.sync_copy(zeros_hbm.at[pl.ds(sid * SUB, SUB)], buf_v)
    pltpu.sync_copy(buf_v, acc0_sh.at[pl.ds(sid * SUB, SUB)])
    pltpu.sync_copy(buf_v, acc1_sh.at[pl.ds(sid * SUB, SUB)])
    plsc.subcore_barrier()
    c0, nch = _worker_chunks(wid, KJM)

    def step(t, carry):
        p = t % 2
        pltpu.sync_copy(e3_hbm.at[pl.ds((c0 + t) * KJM, KJM)], ev)

        @pl.when(t >= 2)
        def _drain():
            pltpu.make_async_copy(zb_hbm, val0_v.at[0], sem).wait()
            pltpu.make_async_copy(zb_hbm, val1_v.at[0], sem).wait()

        for j in range(KJM):
            for k in range(128 // 16):
                idx = ev[j, 0, pl.ds(k * 16, 16)]
                w = plsc.load_gather(upt_v, [idx])
                bf = plsc.bitcast(w, jnp.bfloat16)
                v0, v1 = plsc.unpack(bf, format=plsc.PackFormat.INTERLEAVED)
                o = j * 128 + k * 16
                val0_v[p, pl.ds(o, 16)] = v0
                val1_v[p, pl.ds(o, 16)] = v1
                dstv[p, pl.ds(o, 16)] = ev[j, 1, pl.ds(k * 16, 16)]
        pltpu.async_copy(val0_v.at[p], acc0_sh.at[dstv.at[p]], sem, add=True)
        pltpu.async_copy(val1_v.at[p], acc1_sh.at[dstv.at[p]], sem, add=True)
        return carry

    lax.fori_loop(0, nch, step, 0)
    for _ in range(4):
        pltpu.make_async_copy(zb_hbm, val0_v.at[0], sem).wait()

    @pl.when(wid == NW - 1)
    def _tail():
        pltpu.sync_copy(e3_hbm.at[pl.ds(MNCH * KJM, MTAIL)],
                        ev.at[pl.ds(0, MTAIL)])
        for j in range(MTAIL):
            for k in range(128 // 16):
                idx = ev[j, 0, pl.ds(k * 16, 16)]
                w = plsc.load_gather(upt_v, [idx])
                bf = plsc.bitcast(w, jnp.bfloat16)
                v0, v1 = plsc.unpack(bf, format=plsc.PackFormat.INTERLEAVED)
                tval0_v[pl.ds(k * 16, 16)] = v0
                tval1_v[pl.ds(k * 16, 16)] = v1
            pltpu.sync_copy(tval0_v, acc0_sh.at[ev.at[j, 1]], add=True)
            pltpu.sync_copy(tval1_v, acc1_sh.at[ev.at[j, 1]], add=True)

    plsc.subcore_barrier()
    pltpu.sync_copy(acc0_sh.at[pl.ds(sid * SUB, SUB)], buf_v)
    pltpu.sync_copy(buf_v, out0_hbm.at[pl.ds(cid * NPAD + sid * SUB, SUB)])
    pltpu.sync_copy(acc1_sh.at[pl.ds(sid * SUB, SUB)], buf_v)
    pltpu.sync_copy(buf_v, out1_hbm.at[pl.ds(cid * NPAD + sid * SUB, SUB)])


def _msg_call(e3, up, zeros1):
    o = jax.ShapeDtypeStruct((NC * NPAD,), jnp.float32)
    return pl.kernel(
        _msg_body,
        out_type=(o, o),
        mesh=_mesh(),
        scratch_types=[
            pltpu.VMEM((KJM, 2, 128), jnp.int32),
            pltpu.VMEM((N,), jnp.int32),
            pltpu.VMEM((2, MBIG), jnp.float32),
            pltpu.VMEM((2, MBIG), jnp.float32),
            pltpu.VMEM((2, MBIG), jnp.int32),
            pltpu.VMEM((128,), jnp.float32),
            pltpu.VMEM((128,), jnp.float32),
            pltpu.VMEM((SUB,), jnp.float32),
            pltpu.VMEM_SHARED((NPAD,), jnp.float32),
            pltpu.VMEM_SHARED((NPAD,), jnp.float32),
            pltpu.SemaphoreType.DMA,
        ],
        compiler_params=_SC_PARAMS,
    )(e3, up, zeros1)


def _mid_body(degp_ref, xt_ref, dinv_ref, u0_ref, u1_ref, up_ref):
    deg = degp_ref[0:1, :] + degp_ref[1:2, :] + 1.0
    dinv = lax.rsqrt(deg)
    u0 = dinv * xt_ref[0:1, :]
    u1 = dinv * xt_ref[1:2, :]
    dinv_ref[...] = dinv
    u0_ref[...] = u0
    u1_ref[...] = u1
    b0 = lax.bitcast_convert_type(
        u0.astype(jnp.bfloat16), jnp.uint16).astype(jnp.uint32)
    b1 = lax.bitcast_convert_type(
        u1.astype(jnp.bfloat16), jnp.uint16).astype(jnp.uint32)
    up_ref[...] = lax.bitcast_convert_type(b0 | (b1 << 16), jnp.int32)


def _mid_call(degp2, xt):
    o = jax.ShapeDtypeStruct((1, NPAD), jnp.float32)
    oi = jax.ShapeDtypeStruct((1, NPAD), jnp.int32)
    return pl.pallas_call(
        _mid_body,
        out_shape=[o, o, o, oi],
    )(degp2, xt)


def _head_body(m0, m1, B0, B1, b320, Wf1, bf1, Wf23, bf23,
               fea_out, mu_out, th_out):
    h = (jnp.dot(m0[...], B0[...], preferred_element_type=jnp.float32)
         + jnp.dot(m1[...], B1[...], preferred_element_type=jnp.float32)
         + b320[...])
    h = jnp.maximum(h, 0.0)
    fea = jnp.dot(h, Wf1[...], preferred_element_type=jnp.float32) + bf1[...]
    nrm = jnp.sqrt(jnp.sum(fea * fea, axis=1, keepdims=True))
    fmu = fea / jnp.maximum(nrm, 1e-12)
    s = jnp.dot(fmu, Wf23[...], preferred_element_type=jnp.float32) + bf23[...]
    fea_out[...] = fmu
    mu_out[...] = 1.0 / (1.0 + jnp.exp(-s[:, 0:1]))
    sp = s[:, 1:2]
    th_out[...] = jnp.maximum(sp, 0.0) + jnp.log(1.0 + jnp.exp(-jnp.abs(sp)))


def _head_call(m0, m1, B0, B1, b320, Wf1, bf1, Wf23, bf23):
    nb = 10000 // GB
    bs_g = pl.BlockSpec((GB, 10), lambda i: (i, 0))

    def full(shape):
        return pl.BlockSpec(shape, lambda i: (0,) * len(shape))

    return pl.pallas_call(
        _head_body,
        grid=(nb,),
        in_specs=[bs_g, bs_g,
                  full((10, 320)), full((10, 320)), full((1, 320)),
                  full((320, 256)), full((1, 256)),
                  full((256, 2)), full((1, 2))],
        out_specs=[pl.BlockSpec((GB, 256), lambda i: (i, 0)),
                   pl.BlockSpec((GB, 1), lambda i: (i, 0)),
                   pl.BlockSpec((GB, 1), lambda i: (i, 0))],
        out_shape=[jax.ShapeDtypeStruct((10000, 256), jnp.float32),
                   jax.ShapeDtypeStruct((10000, 1), jnp.float32),
                   jax.ShapeDtypeStruct((10000, 1), jnp.float32)],
    )(m0, m1, B0, B1, b320, Wf1, bf1, Wf23, bf23)


def kernel(x, edge_index, W1, b1, Wf1, bf1, Wf2, bf2, Wf3, bf3):
    e3 = edge_index.astype(jnp.int32).reshape(2, ER, 128).transpose(1, 0, 2)
    zeros1 = jnp.zeros((NPAD,), jnp.float32)

    degp = _deg_call(e3, zeros1).reshape(NC, NPAD)
    xt = jnp.pad(x.T, ((0, 0), (0, NPAD - N)))
    dinv, u0, u1, up = _mid_call(degp, xt)

    agg0p, agg1p = _msg_call(e3, up[0], zeros1)
    m0 = ((agg0p.reshape(NC, NPAD).sum(0) + u0[0])
          * dinv[0])[:N].reshape(10000, 10)
    m1 = ((agg1p.reshape(NC, NPAD).sum(0) + u1[0])
          * dinv[0])[:N].reshape(10000, 10)

    B0 = jnp.kron(jnp.eye(10, dtype=jnp.float32), W1[0:1, :])
    B1 = jnp.kron(jnp.eye(10, dtype=jnp.float32), W1[1:2, :])
    b320 = jnp.tile(b1, 10)[None, :]
    Wf23 = jnp.concatenate([Wf2, Wf3], axis=1)
    bf23 = jnp.concatenate([bf2, bf3])[None, :]

    fea_mu, mu, th = _head_call(m0, m1, B0, B1, b320,
                                Wf1, bf1[None, :], Wf23, bf23)
    return (fea_mu, mu[:, 0], th[:, 0])
